# baseline (speedup 1.0000x reference)
import functools
import jax
import jax.numpy as jnp
from jax import lax
from jax.experimental import pallas as pl
from jax.experimental.pallas import tpu as pltpu
from jax.experimental.pallas import tpu_sc as plsc

N = 10000
E = 320000
D = 128

_NC = 2
_NS = 16
_NW = _NC * _NS

_GC = 64
_NCH = E // _GC
_CPW = _NCH // _NW
_XTRA = _NCH - _CPW * _NW

_NSH = 2
_NG = 16
_SC4 = 512
_NC4 = E // _SC4

_NEG = float("-inf")


def _dense1_body(x_ref, pos_ref, Wh1_ref, bh1_ref, Wh2_ref, bh2_ref,
                 W3a_ref, W128a_ref, W3t_ref, W128t_ref, bf1a_ref, bf1t_ref,
                 a_ref, b_ref, u_ref, v_ref):
    x = x_ref[...]
    pos = pos_ref[...]
    h = jnp.maximum(x @ Wh1_ref[...] + bh1_ref[...], 0.0)
    delta = h @ Wh2_ref[...] + bh2_ref[...]
    q = delta - pos
    a_ref[...] = pos @ W3a_ref[...] + x @ W128a_ref[...]
    b_ref[...] = q @ W3a_ref[...] + bf1a_ref[...]
    cd = (((0,), (1,)), ((), ()))
    u_ref[...] = (lax.dot_general(W3t_ref[...], pos, cd)
                  + lax.dot_general(W128t_ref[...], x, cd))
    v_ref[...] = lax.dot_general(W3t_ref[...], q, cd) + bf1t_ref[...]


def _gather_body(a_hbm, b_hbm, u_hbm, v_hbm, src_hbm, dst_hbm,
                 h_hbm, ht_hbm, dup_hbm,
                 src0, dst0, ra0, rb0, src1, dst1, ra1, rb1,
                 ro0, htb0, dupb0, ro1, htb1, dupb1,
                 uv, vv, sem0, sem1, semo0, semo1):
    wid = lax.axis_index("s") * _NC + lax.axis_index("c")

    lanes = jnp.arange(16, dtype=jnp.int32)
    shl = jnp.maximum(lanes - 1, 0)
    nfirst = lanes >= 1

    pltpu.sync_copy(u_hbm, uv)
    pltpu.sync_copy(v_hbm, vv)

    bufs = ((src0, dst0, ra0, rb0, sem0), (src1, dst1, ra1, rb1, sem1))
    obufs = ((ro0, htb0, dupb0, semo0), (ro1, htb1, dupb1, semo1))

    def issue(off, p):
        sv, dv, ra, rb, sem = bufs[p]
        pltpu.sync_copy(src_hbm.at[pl.ds(off, _GC)], sv)
        pltpu.sync_copy(dst_hbm.at[pl.ds(off, _GC)], dv)
        pltpu.async_copy(a_hbm.at[sv], ra, sem)
        pltpu.async_copy(b_hbm.at[dv], rb, sem)

    def wait(p):
        sv, dv, ra, rb, sem = bufs[p]
        pltpu.make_async_copy(a_hbm.at[sv], ra, sem).wait()
        pltpu.make_async_copy(b_hbm.at[dv], rb, sem).wait()

    def wait_out(p):
        ro, htb, dupb, semo = obufs[p]
        pltpu.make_async_copy(ro, h_hbm.at[pl.ds(0, _GC)], semo).wait()
        pltpu.make_async_copy(htb, ht_hbm.at[:, pl.ds(0, _GC)], semo).wait()
        pltpu.make_async_copy(dupb, dup_hbm.at[pl.ds(0, _GC)], semo).wait()

    def compute(off, p):
        sv, dv, ra, rb, sem = bufs[p]
        ro, htb, dupb, semo = obufs[p]

        def addrow(r, c2):
            for k in range(D // 16):
                sl = pl.ds(k * 16, 16)
                ro[r, sl] = ra[r, sl] + rb[r, sl]
            return c2

        lax.fori_loop(0, _GC, addrow, 0)

        def tailgrp(gi, c2):
            r0 = gi * 16
            svv = sv[pl.ds(r0, 16)]
            dvv = dv[pl.ds(r0, 16)]
            for c in range(3):
                tu = plsc.load_gather(uv, [svv + c * N])
                tv = plsc.load_gather(vv, [dvv + c * N])
                htb[c, pl.ds(r0, 16)] = tu + tv
            srt = lax.sort(dvv)
            adj = jnp.take_along_axis(srt, shl, axis=0)
            has = jnp.max(jnp.where((srt == adj) & nfirst, 1, 0))
            dupb[pl.ds(r0, 16)] = jnp.full((16,), has, jnp.int32)
            return c2

        lax.fori_loop(0, _GC // 16, tailgrp, 0)

        pltpu.async_copy(ro, h_hbm.at[pl.ds(off, _GC)], semo)
        pltpu.async_copy(htb, ht_hbm.at[:, pl.ds(off, _GC)], semo)
        pltpu.async_copy(dupb, dup_hbm.at[pl.ds(off, _GC)], semo)

    coff = lambda i: (wid + i * _NW) * _GC

    issue(coff(0), 0)

    def pair(j, carry):
        issue(coff(2 * j + 1), 1)
        wait(0)

        @pl.when(j > 0)
        def _():
            wait_out(0)

        compute(coff(2 * j), 0)

        @pl.when(j < _CPW // 2 - 1)
        def _():
            issue(coff(2 * j + 2), 0)

        wait(1)

        @pl.when(j > 0)
        def _():
            wait_out(1)

        compute(coff(2 * j + 1), 1)
        return carry

    lax.fori_loop(0, _CPW // 2, pair, 0)

    @pl.when(wid < _XTRA)
    def _leftover():
        off = (_CPW * _NW + wid) * _GC
        issue(off, 0)
        wait(0)
        wait_out(0)
        compute(off, 0)

    wait_out(0)
    wait_out(1)


def _edge_mlp_body(h_ref, ht_ref, Wa_ref, Wt_ref, b_ref, o_ref):
    h = jnp.maximum(h_ref[...], 0.0)
    ht = jnp.maximum(ht_ref[...], 0.0)
    o_ref[...] = (lax.dot_general(Wa_ref[...], h, (((0,), (1,)), ((), ())))
                  + lax.dot_general(Wt_ref[...], ht, (((0,), (0,)), ((), ())))
                  + b_ref[...])


def _scatmax_body(e_hbm, dst_hbm, dup_hbm, out_hbm,
                  eb0, db0, up0, eb1, db1, up1,
                  a0, a1, a2, a3, a4, a5, a6, a7, sem0, sem1):
    wid = lax.axis_index("s") * _NC + lax.axis_index("c")
    g = wid % _NG
    sh = wid // _NG

    lanes = jnp.arange(16, dtype=jnp.int32)
    accs = (a0, a1, a2, a3, a4, a5, a6, a7)

    neg = jnp.full((16,), _NEG, jnp.float32)

    def initloop(j, c):
        for r in range(8):
            accs[r][pl.ds(j * 16, 16)] = neg
        return c

    lax.fori_loop(0, N // 16, initloop, 0)

    bufs = ((eb0, db0, up0, sem0), (eb1, db1, up1, sem1))

    def issue(ci, p):
        eb, db, up, sem = bufs[p]
        eoff = ci * _SC4
        pltpu.async_copy(dst_hbm.at[pl.ds(eoff, _SC4)], db, sem)
        pltpu.async_copy(dup_hbm.at[pl.ds(eoff, _SC4)], up, sem)
        pltpu.async_copy(e_hbm.at[pl.ds(8 * g, 8), pl.ds(eoff, _SC4)], eb, sem)

    def wait(p):
        eb, db, up, sem = bufs[p]
        pltpu.make_async_copy(dst_hbm.at[pl.ds(0, _SC4)], db, sem).wait()
        pltpu.make_async_copy(dup_hbm.at[pl.ds(0, _SC4)], up, sem).wait()
        pltpu.make_async_copy(
            e_hbm.at[pl.ds(8 * g, 8), pl.ds(0, _SC4)], eb, sem).wait()

    def compute(p):
        eb, db, up, sem = bufs[p]
        ngrp = _SC4 // 16

        def orstep(k, m):
            acc_m = m
            for q in range(4):
                acc_m = jnp.maximum(acc_m, up[pl.ds((k * 4 + q) * 16, 16)])
            return acc_m

        any_dup = jnp.max(lax.fori_loop(
            0, ngrp // 4, orstep, jnp.zeros((16,), jnp.int32)))

        @pl.when(any_dup == 0)
        def _clean():
            def grp5(blk, c2):
                for q in range(8):
                    r0 = (blk * 8 + q) * 16
                    dv = db[pl.ds(r0, 16)]
                    for r in range(8):
                        val = eb[r, pl.ds(r0, 16)]
                        cur = plsc.load_gather(accs[r], [dv])
                        plsc.store_scatter(accs[r], [dv],
                                           jnp.maximum(cur, val))
                return c2

            lax.fori_loop(0, ngrp // 8, grp5, 0)

        @pl.when(any_dup != 0)
        def _dirty():
            def grpstep(blk, c2):
                r0 = blk * 16
                dv = db[pl.ds(r0, 16)]
                has = up[pl.ds(r0, 16)][0]

                @pl.when(has == 0)
                def _fast():
                    for r in range(8):
                        val = eb[r, pl.ds(r0, 16)]
                        cur = plsc.load_gather(accs[r], [dv])
                        plsc.store_scatter(accs[r], [dv],
                                           jnp.maximum(cur, val))

                @pl.when(has != 0)
                def _slow():
                    for r in range(8):
                        val = eb[r, pl.ds(r0, 16)]
                        for i in range(16):
                            cur = plsc.load_gather(accs[r], [dv])
                            plsc.store_scatter(accs[r], [dv],
                                               jnp.maximum(cur, val),
                                               mask=lanes == i)
                return c2

            lax.fori_loop(0, ngrp, grpstep, 0)

    npair = 156

    issue(sh, 0)

    def pair(j, carry):
        issue(4 * j + 2 + sh, 1)
        wait(0)
        compute(0)

        @pl.when(j < npair - 1)
        def _():
            issue(4 * j + 4 + sh, 0)

        wait(1)
        compute(1)
        return carry

    lax.fori_loop(0, npair, pair, 0)

    @pl.when(sh == 0)
    def _leftover():
        issue(_NC4 - 1, 0)
        wait(0)
        compute(0)

    for r in range(8):
        pltpu.sync_copy(accs[r], out_hbm.at[sh, 8 * g + r])


def _dense2_body(at_ref, x_ref, Wg1_ref, bg1_ref, Wg2_ref, bg2_ref, o_ref):
    a = jnp.maximum(at_ref[0], at_ref[1])
    a = jnp.where(jnp.isfinite(a), a, 0.0)
    h1 = lax.dot_general(a, Wg1_ref[...], (((0,), (0,)), ((), ())))
    h = jnp.maximum(h1 + bg1_ref[...], 0.0)
    o_ref[...] = h @ Wg2_ref[...] + bg2_ref[...] + x_ref[...]


def kernel(x, pos, edge_index, Wh1, bh1, Wh2, bh2, Wf1, bf1, Wf2, bf2, Wg1, bg1, Wg2, bg2):
    src = edge_index[0]
    dst = edge_index[1]
    W3a = Wf1[:3, :D]
    W128a = Wf1[3:, :D]
    W3t = jnp.zeros((3, 4), jnp.float32).at[:, :3].set(Wf1[:3, D:])
    W128t = jnp.zeros((D, 4), jnp.float32).at[:, :3].set(Wf1[3:, D:])
    bf1a = bf1[:D]
    bf1t = jnp.zeros((4,), jnp.float32).at[:3].set(bf1[D:])
    Wf2a = Wf2[:D]
    Wf2t = jnp.zeros((4, D), jnp.float32).at[:3].set(Wf2[D:])

    a128, b128, u_tab, v_tab = pl.pallas_call(
        _dense1_body,
        out_shape=[
            jax.ShapeDtypeStruct((N, D), jnp.float32),
            jax.ShapeDtypeStruct((N, D), jnp.float32),
            jax.ShapeDtypeStruct((4, N), jnp.float32),
            jax.ShapeDtypeStruct((4, N), jnp.float32),
        ],
    )(x, pos, Wh1, bh1[None, :], Wh2, bh2[None, :],
      W3a, W128a, W3t, W128t, bf1a[None, :], bf1t[:, None])

    mesh = plsc.VectorSubcoreMesh(core_axis_name="c", subcore_axis_name="s")
    sc_params = pltpu.CompilerParams(
        use_tc_tiling_on_sc=False, needs_layout_passes=False)
    h128, ht, dupm = pl.kernel(
        _gather_body,
        mesh=mesh,
        compiler_params=sc_params,
        out_type=[
            jax.ShapeDtypeStruct((E, D), jnp.float32),
            jax.ShapeDtypeStruct((4, E), jnp.float32),
            jax.ShapeDtypeStruct((E,), jnp.int32),
        ],
        scratch_types=[
            pltpu.VMEM((_GC,), jnp.int32),
            pltpu.VMEM((_GC,), jnp.int32),
            pltpu.VMEM((_GC, D), jnp.float32),
            pltpu.VMEM((_GC, D), jnp.float32),
            pltpu.VMEM((_GC,), jnp.int32),
            pltpu.VMEM((_GC,), jnp.int32),
            pltpu.VMEM((_GC, D), jnp.float32),
            pltpu.VMEM((_GC, D), jnp.float32),
            pltpu.VMEM((_GC, D), jnp.float32),
            pltpu.VMEM((4, _GC), jnp.float32),
            pltpu.VMEM((_GC,), jnp.int32),
            pltpu.VMEM((_GC, D), jnp.float32),
            pltpu.VMEM((4, _GC), jnp.float32),
            pltpu.VMEM((_GC,), jnp.int32),
            pltpu.VMEM((3 * N,), jnp.float32),
            pltpu.VMEM((3 * N,), jnp.float32),
            pltpu.SemaphoreType.DMA,
            pltpu.SemaphoreType.DMA,
            pltpu.SemaphoreType.DMA,
            pltpu.SemaphoreType.DMA,
        ],
    )(a128, b128, u_tab[:3].reshape(3 * N), v_tab[:3].reshape(3 * N), src, dst)

    EB = 1280
    e_arr = pl.pallas_call(
        _edge_mlp_body,
        grid=(E // EB,),
        in_specs=[
            pl.BlockSpec((EB, D), lambda i: (i, 0)),
            pl.BlockSpec((4, EB), lambda i: (0, i)),
            pl.BlockSpec((D, D), lambda i: (0, 0)),
            pl.BlockSpec((4, D), lambda i: (0, 0)),
            pl.BlockSpec((D, 1), lambda i: (0, 0)),
        ],
        out_specs=pl.BlockSpec((D, EB), lambda i: (0, i)),
        out_shape=jax.ShapeDtypeStruct((D, E), jnp.float32),
    )(h128, ht, Wf2a, Wf2t, bf2[:, None])

    aggr_t = pl.kernel(
        _scatmax_body,
        mesh=mesh,
        compiler_params=pltpu.CompilerParams(needs_layout_passes=False),
        out_type=jax.ShapeDtypeStruct((_NSH, D, N), jnp.float32),
        scratch_types=[
            pltpu.VMEM((8, _SC4), jnp.float32),
            pltpu.VMEM((_SC4,), jnp.int32),
            pltpu.VMEM((_SC4,), jnp.int32),
            pltpu.VMEM((8, _SC4), jnp.float32),
            pltpu.VMEM((_SC4,), jnp.int32),
            pltpu.VMEM((_SC4,), jnp.int32),
        ] + [pltpu.VMEM((N,), jnp.float32)] * 8 + [
            pltpu.SemaphoreType.DMA,
            pltpu.SemaphoreType.DMA,
        ],
    )(e_arr, dst, dupm)

    out = pl.pallas_call(
        _dense2_body,
        out_shape=jax.ShapeDtypeStruct((N, D), jnp.float32),
    )(aggr_t, x, Wg1, bg1[None, :], Wg2, bg2[None, :])
    return out


# speedup vs baseline: 1.0037x; 1.0037x over previous
import functools
import jax
import jax.numpy as jnp
from jax import lax
from jax.experimental import pallas as pl
from jax.experimental.pallas import tpu as pltpu
from jax.experimental.pallas import tpu_sc as plsc

N = 10000
E = 320000
D = 128

_NC = 2
_NS = 16
_NW = _NC * _NS

_GC = 64
_NCH = E // _GC
_CPW = _NCH // _NW
_XTRA = _NCH - _CPW * _NW

_NSH = 2
_NG = 16
_SC4 = 512
_NC4 = E // _SC4

_NEG = float("-inf")


def _dense1_body(x_ref, pos_ref, Wh1_ref, bh1_ref, Wh2_ref, bh2_ref,
                 W3a_ref, W128a_ref, W3t_ref, W128t_ref, bf1a_ref, bf1t_ref,
                 a_ref, b_ref, u_ref, v_ref):
    x = x_ref[...]
    pos = pos_ref[...]
    h = jnp.maximum(x @ Wh1_ref[...] + bh1_ref[...], 0.0)
    delta = h @ Wh2_ref[...] + bh2_ref[...]
    q = delta - pos
    a_ref[...] = pos @ W3a_ref[...] + x @ W128a_ref[...]
    b_ref[...] = q @ W3a_ref[...] + bf1a_ref[...]
    cd = (((0,), (1,)), ((), ()))
    u_ref[...] = (lax.dot_general(W3t_ref[...], pos, cd)
                  + lax.dot_general(W128t_ref[...], x, cd))
    v_ref[...] = lax.dot_general(W3t_ref[...], q, cd) + bf1t_ref[...]


def _gather_body(a_hbm, b_hbm, u_hbm, v_hbm, src_hbm, dst_hbm,
                 h_hbm, ht_hbm, dup_hbm,
                 src0, dst0, ra0, rb0, src1, dst1, ra1, rb1,
                 ro0, htb0, dupb0, ro1, htb1, dupb1,
                 uv, vv, sem0, sem1, semo0, semo1):
    wid = lax.axis_index("s") * _NC + lax.axis_index("c")

    lanes = jnp.arange(16, dtype=jnp.int32)
    shl = jnp.maximum(lanes - 1, 0)
    nfirst = lanes >= 1

    pltpu.sync_copy(u_hbm, uv)
    pltpu.sync_copy(v_hbm, vv)

    bufs = ((src0, dst0, ra0, rb0, sem0), (src1, dst1, ra1, rb1, sem1))
    obufs = ((ro0, htb0, dupb0, semo0), (ro1, htb1, dupb1, semo1))

    def issue(off, p):
        sv, dv, ra, rb, sem = bufs[p]
        pltpu.sync_copy(src_hbm.at[pl.ds(off, _GC)], sv)
        pltpu.sync_copy(dst_hbm.at[pl.ds(off, _GC)], dv)
        pltpu.async_copy(a_hbm.at[sv], ra, sem)
        pltpu.async_copy(b_hbm.at[dv], rb, sem)

    def wait(p):
        sv, dv, ra, rb, sem = bufs[p]
        pltpu.make_async_copy(a_hbm.at[sv], ra, sem).wait()
        pltpu.make_async_copy(b_hbm.at[dv], rb, sem).wait()

    def wait_out(p):
        ro, htb, dupb, semo = obufs[p]
        pltpu.make_async_copy(ro, h_hbm.at[pl.ds(0, _GC)], semo).wait()
        pltpu.make_async_copy(htb, ht_hbm.at[:, pl.ds(0, _GC)], semo).wait()
        pltpu.make_async_copy(dupb, dup_hbm.at[pl.ds(0, _GC)], semo).wait()

    def compute(off, p):
        sv, dv, ra, rb, sem = bufs[p]
        ro, htb, dupb, semo = obufs[p]

        def addrow(r, c2):
            for k in range(D // 16):
                sl = pl.ds(k * 16, 16)
                ro[r, sl] = ra[r, sl] + rb[r, sl]
            return c2

        lax.fori_loop(0, _GC, addrow, 0)

        def tailgrp(gi, c2):
            r0 = gi * 16
            svv = sv[pl.ds(r0, 16)]
            dvv = dv[pl.ds(r0, 16)]
            for c in range(3):
                tu = plsc.load_gather(uv, [svv + c * N])
                tv = plsc.load_gather(vv, [dvv + c * N])
                htb[c, pl.ds(r0, 16)] = tu + tv
            srt = lax.sort(dvv)
            adj = jnp.take_along_axis(srt, shl, axis=0)
            has = jnp.max(jnp.where((srt == adj) & nfirst, 1, 0))
            dupb[pl.ds(r0, 16)] = jnp.full((16,), has, jnp.int32)
            return c2

        lax.fori_loop(0, _GC // 16, tailgrp, 0)

        pltpu.async_copy(ro, h_hbm.at[pl.ds(off, _GC)], semo)
        pltpu.async_copy(htb, ht_hbm.at[:, pl.ds(off, _GC)], semo)
        pltpu.async_copy(dupb, dup_hbm.at[pl.ds(off, _GC)], semo)

    coff = lambda i: (wid + i * _NW) * _GC

    issue(coff(0), 0)

    def pair(j, carry):
        issue(coff(2 * j + 1), 1)
        wait(0)

        @pl.when(j > 0)
        def _():
            wait_out(0)

        compute(coff(2 * j), 0)

        @pl.when(j < _CPW // 2 - 1)
        def _():
            issue(coff(2 * j + 2), 0)

        wait(1)

        @pl.when(j > 0)
        def _():
            wait_out(1)

        compute(coff(2 * j + 1), 1)
        return carry

    lax.fori_loop(0, _CPW // 2, pair, 0)

    @pl.when(wid < _XTRA)
    def _leftover():
        off = (_CPW * _NW + wid) * _GC
        issue(off, 0)
        wait(0)
        wait_out(0)
        compute(off, 0)

    wait_out(0)
    wait_out(1)


def _edge_mlp_body(h_ref, ht_ref, Wa_ref, Wt_ref, b_ref, o_ref):
    h = jnp.maximum(h_ref[...], 0.0)
    ht = jnp.maximum(ht_ref[...], 0.0)
    o_ref[...] = (lax.dot_general(Wa_ref[...], h, (((0,), (1,)), ((), ())))
                  + lax.dot_general(Wt_ref[...], ht, (((0,), (0,)), ((), ())))
                  + b_ref[...])


def _scatmax_body(e_hbm, dst_hbm, dup_hbm, out_hbm,
                  eb0, db0, up0, eb1, db1, up1,
                  a0, a1, a2, a3, a4, a5, a6, a7, sem0, sem1):
    wid = lax.axis_index("s") * _NC + lax.axis_index("c")
    g = wid % _NG
    sh = wid // _NG

    lanes = jnp.arange(16, dtype=jnp.int32)
    accs = (a0, a1, a2, a3, a4, a5, a6, a7)

    neg = jnp.full((16,), _NEG, jnp.float32)

    def initloop(j, c):
        for r in range(8):
            accs[r][pl.ds(j * 16, 16)] = neg
        return c

    lax.fori_loop(0, N // 16, initloop, 0)

    bufs = ((eb0, db0, up0, sem0), (eb1, db1, up1, sem1))

    def issue(ci, p):
        eb, db, up, sem = bufs[p]
        eoff = ci * _SC4
        pltpu.async_copy(dst_hbm.at[pl.ds(eoff, _SC4)], db, sem)
        pltpu.async_copy(dup_hbm.at[pl.ds(eoff, _SC4)], up, sem)
        pltpu.async_copy(e_hbm.at[pl.ds(8 * g, 8), pl.ds(eoff, _SC4)], eb, sem)

    def wait(p):
        eb, db, up, sem = bufs[p]
        pltpu.make_async_copy(dst_hbm.at[pl.ds(0, _SC4)], db, sem).wait()
        pltpu.make_async_copy(dup_hbm.at[pl.ds(0, _SC4)], up, sem).wait()
        pltpu.make_async_copy(
            e_hbm.at[pl.ds(8 * g, 8), pl.ds(0, _SC4)], eb, sem).wait()

    def compute(p):
        eb, db, up, sem = bufs[p]
        ngrp = _SC4 // 16

        def orstep(k, m):
            acc_m = m
            for q in range(4):
                acc_m = jnp.maximum(acc_m, up[pl.ds((k * 4 + q) * 16, 16)])
            return acc_m

        any_dup = jnp.max(lax.fori_loop(
            0, ngrp // 4, orstep, jnp.zeros((16,), jnp.int32)))

        @pl.when(any_dup == 0)
        def _clean():
            def grp5(blk, c2):
                for q in range(4):
                    r0 = (blk * 4 + q) * 16
                    dv = db[pl.ds(r0, 16)]
                    for r in range(8):
                        val = eb[r, pl.ds(r0, 16)]
                        cur = plsc.load_gather(accs[r], [dv])
                        plsc.store_scatter(accs[r], [dv],
                                           jnp.maximum(cur, val))
                return c2

            lax.fori_loop(0, ngrp // 4, grp5, 0)

        @pl.when(any_dup != 0)
        def _dirty():
            def grpstep(blk, c2):
                r0 = blk * 16
                dv = db[pl.ds(r0, 16)]
                has = up[pl.ds(r0, 16)][0]

                @pl.when(has == 0)
                def _fast():
                    for r in range(8):
                        val = eb[r, pl.ds(r0, 16)]
                        cur = plsc.load_gather(accs[r], [dv])
                        plsc.store_scatter(accs[r], [dv],
                                           jnp.maximum(cur, val))

                @pl.when(has != 0)
                def _slow():
                    for r in range(8):
                        val = eb[r, pl.ds(r0, 16)]
                        for i in range(16):
                            cur = plsc.load_gather(accs[r], [dv])
                            plsc.store_scatter(accs[r], [dv],
                                               jnp.maximum(cur, val),
                                               mask=lanes == i)
                return c2

            lax.fori_loop(0, ngrp, grpstep, 0)

    npair = 156

    issue(sh, 0)

    def pair(j, carry):
        issue(4 * j + 2 + sh, 1)
        wait(0)
        compute(0)

        @pl.when(j < npair - 1)
        def _():
            issue(4 * j + 4 + sh, 0)

        wait(1)
        compute(1)
        return carry

    lax.fori_loop(0, npair, pair, 0)

    @pl.when(sh == 0)
    def _leftover():
        issue(_NC4 - 1, 0)
        wait(0)
        compute(0)

    for r in range(8):
        pltpu.sync_copy(accs[r], out_hbm.at[sh, 8 * g + r])


def _dense2_body(at_ref, x_ref, Wg1_ref, bg1_ref, Wg2_ref, bg2_ref, o_ref):
    a = jnp.maximum(at_ref[0], at_ref[1])
    a = jnp.where(jnp.isfinite(a), a, 0.0)
    h1 = lax.dot_general(a, Wg1_ref[...], (((0,), (0,)), ((), ())))
    h = jnp.maximum(h1 + bg1_ref[...], 0.0)
    o_ref[...] = h @ Wg2_ref[...] + bg2_ref[...] + x_ref[...]


def kernel(x, pos, edge_index, Wh1, bh1, Wh2, bh2, Wf1, bf1, Wf2, bf2, Wg1, bg1, Wg2, bg2):
    src = edge_index[0]
    dst = edge_index[1]
    W3a = Wf1[:3, :D]
    W128a = Wf1[3:, :D]
    W3t = jnp.zeros((3, 4), jnp.float32).at[:, :3].set(Wf1[:3, D:])
    W128t = jnp.zeros((D, 4), jnp.float32).at[:, :3].set(Wf1[3:, D:])
    bf1a = bf1[:D]
    bf1t = jnp.zeros((4,), jnp.float32).at[:3].set(bf1[D:])
    Wf2a = Wf2[:D]
    Wf2t = jnp.zeros((4, D), jnp.float32).at[:3].set(Wf2[D:])

    a128, b128, u_tab, v_tab = pl.pallas_call(
        _dense1_body,
        out_shape=[
            jax.ShapeDtypeStruct((N, D), jnp.float32),
            jax.ShapeDtypeStruct((N, D), jnp.float32),
            jax.ShapeDtypeStruct((4, N), jnp.float32),
            jax.ShapeDtypeStruct((4, N), jnp.float32),
        ],
    )(x, pos, Wh1, bh1[None, :], Wh2, bh2[None, :],
      W3a, W128a, W3t, W128t, bf1a[None, :], bf1t[:, None])

    mesh = plsc.VectorSubcoreMesh(core_axis_name="c", subcore_axis_name="s")
    sc_params = pltpu.CompilerParams(
        use_tc_tiling_on_sc=False, needs_layout_passes=False)
    h128, ht, dupm = pl.kernel(
        _gather_body,
        mesh=mesh,
        compiler_params=sc_params,
        out_type=[
            jax.ShapeDtypeStruct((E, D), jnp.float32),
            jax.ShapeDtypeStruct((4, E), jnp.float32),
            jax.ShapeDtypeStruct((E,), jnp.int32),
        ],
        scratch_types=[
            pltpu.VMEM((_GC,), jnp.int32),
            pltpu.VMEM((_GC,), jnp.int32),
            pltpu.VMEM((_GC, D), jnp.float32),
            pltpu.VMEM((_GC, D), jnp.float32),
            pltpu.VMEM((_GC,), jnp.int32),
            pltpu.VMEM((_GC,), jnp.int32),
            pltpu.VMEM((_GC, D), jnp.float32),
            pltpu.VMEM((_GC, D), jnp.float32),
            pltpu.VMEM((_GC, D), jnp.float32),
            pltpu.VMEM((4, _GC), jnp.float32),
            pltpu.VMEM((_GC,), jnp.int32),
            pltpu.VMEM((_GC, D), jnp.float32),
            pltpu.VMEM((4, _GC), jnp.float32),
            pltpu.VMEM((_GC,), jnp.int32),
            pltpu.VMEM((3 * N,), jnp.float32),
            pltpu.VMEM((3 * N,), jnp.float32),
            pltpu.SemaphoreType.DMA,
            pltpu.SemaphoreType.DMA,
            pltpu.SemaphoreType.DMA,
            pltpu.SemaphoreType.DMA,
        ],
    )(a128, b128, u_tab[:3].reshape(3 * N), v_tab[:3].reshape(3 * N), src, dst)

    EB = 1280
    e_arr = pl.pallas_call(
        _edge_mlp_body,
        grid=(E // EB,),
        in_specs=[
            pl.BlockSpec((EB, D), lambda i: (i, 0)),
            pl.BlockSpec((4, EB), lambda i: (0, i)),
            pl.BlockSpec((D, D), lambda i: (0, 0)),
            pl.BlockSpec((4, D), lambda i: (0, 0)),
            pl.BlockSpec((D, 1), lambda i: (0, 0)),
        ],
        out_specs=pl.BlockSpec((D, EB), lambda i: (0, i)),
        out_shape=jax.ShapeDtypeStruct((D, E), jnp.float32),
    )(h128, ht, Wf2a, Wf2t, bf2[:, None])

    aggr_t = pl.kernel(
        _scatmax_body,
        mesh=mesh,
        compiler_params=pltpu.CompilerParams(needs_layout_passes=False),
        out_type=jax.ShapeDtypeStruct((_NSH, D, N), jnp.float32),
        scratch_types=[
            pltpu.VMEM((8, _SC4), jnp.float32),
            pltpu.VMEM((_SC4,), jnp.int32),
            pltpu.VMEM((_SC4,), jnp.int32),
            pltpu.VMEM((8, _SC4), jnp.float32),
            pltpu.VMEM((_SC4,), jnp.int32),
            pltpu.VMEM((_SC4,), jnp.int32),
        ] + [pltpu.VMEM((N,), jnp.float32)] * 8 + [
            pltpu.SemaphoreType.DMA,
            pltpu.SemaphoreType.DMA,
        ],
    )(e_arr, dst, dupm)

    out = pl.pallas_call(
        _dense2_body,
        out_shape=jax.ShapeDtypeStruct((N, D), jnp.float32),
    )(aggr_t, x, Wg1, bg1[None, :], Wg2, bg2[None, :])
    return out
